# baseline (speedup 1.0000x reference)
import jax
import jax.numpy as jnp
from jax.experimental import pallas as pl

_N = 8192
_D = 32
_R = 64
_EPS = 1e-5
_CLAMP = 64e-6


def _body(z_ref, w4, b4, g, h_m, wa, r64, e_m, a1, b_m, bo_ref, mask_ref,
          w1g, b1x, g1p, be1p, w2t, b2p, g2m, be2, out_ref):
    Z = z_ref[...]
    f32 = jnp.float32
    dot = lambda a, b: jax.lax.dot_general(
        a, b, (((1,), (0,)), ((), ())), preferred_element_type=f32)

    QKVV = dot(Z, w4[...]) + b4[...]
    NRM = dot(QKVV * QKVV, h_m[...])
    P = dot(QKVV, g[...])
    T = jnp.minimum(jax.lax.rsqrt(NRM), 1e6)
    PHI = jnp.exp(P * T)

    S_full = jax.lax.dot_general(
        PHI, QKVV, (((0,), (0,)), ((), ())), preferred_element_type=f32)
    M2 = dot(r64[...], dot(S_full, wa[...]))

    U = dot(PHI, M2)
    Dn = dot(U, b_m[...])
    t = (1.0 / jnp.maximum(Dn, _CLAMP)) * mask_ref[...]
    X = dot(Z, e_m[...]) + U * t + bo_ref[...]

    SX = dot(X, a1[...])
    SXX = dot(X * X, a1[...])
    XN = (X - SX) * jax.lax.rsqrt(SXX - SX * SX + _EPS)

    hid = jnp.maximum(dot(XN, w1g[...]) + b1x[...], 0.0)
    Z1 = XN * g1p[...] + be1p[...]
    Y = Z1 + dot(hid, w2t[...]) + b2p[...]

    SY = dot(Y, a1[...])
    SYY = dot(Y * Y, a1[...])
    YN = (Y - SY) * jax.lax.rsqrt(SYY - SY * SY + _EPS)

    out_ref[...] = dot(YN, g2m[...]) + be2[...]


@jax.jit
def kernel(Z, Wq, bq, Wk, bk, Wv, bv, Wo, bo, W1, b1, W2, b2,
           g1, beta1, g2, beta2, omega):
    f32 = jnp.float32
    zed = lambda *s: jnp.zeros(s, f32)

    W4 = jnp.concatenate([Wq.T, Wk.T, Wv.T, zed(_D, _D)], axis=1)
    b4 = jnp.concatenate([bq, bk, bv, zed(_D)]).reshape(1, 128)

    G = zed(128, 128).at[0:_D, 0:_R].set(omega).at[_D:2 * _D, _R:128].set(omega)
    H = zed(128, 128).at[0:_D, 0:_R].set(1.0).at[_D:2 * _D, _R:128].set(1.0)

    WA = zed(128, 128).at[2 * _D:3 * _D, 0:_D].set(Wo.T)
    WA = WA.at[2 * _D:3 * _D, _D].set(1.0)
    R64 = zed(128, 128).at[jnp.arange(_R), jnp.arange(_R) + _R].set(1.0)

    E = zed(_D, 128).at[0:_D, 0:_D].set(jnp.eye(_D, dtype=f32))
    A1 = zed(128, 128).at[0:_D, 0:_D].set(1.0 / _D)
    B = zed(128, 128).at[_D, 0:_D].set(1.0)
    bo128 = jnp.concatenate([bo, zed(96)]).reshape(1, 128)
    mask = zed(1, 128).at[0, 0:_D].set(1.0)

    W1g = jnp.concatenate([W1.T * g1[:, None], zed(96, 128)], axis=0)
    b1x = (beta1 @ W1.T + b1).reshape(1, 128)
    g1p = jnp.concatenate([g1, zed(96)]).reshape(1, 128)
    be1p = jnp.concatenate([beta1, zed(96)]).reshape(1, 128)
    W2t = jnp.concatenate([W2.T, zed(128, 96)], axis=1)
    b2p = jnp.concatenate([b2, zed(96)]).reshape(1, 128)
    G2 = zed(128, _D).at[jnp.arange(_D), jnp.arange(_D)].set(g2)
    be2 = beta2.reshape(1, _D)

    args = (Z, W4, b4, G, H, WA, R64, E, A1, B, bo128, mask,
            W1g, b1x, g1p, be1p, W2t, b2p, G2, be2)
    return pl.pallas_call(
        _body,
        out_shape=jax.ShapeDtypeStruct((_N, _D), f32),
    )(*args)


# speedup vs baseline: 2.1607x; 2.1607x over previous
import jax
import jax.numpy as jnp
from jax.experimental import pallas as pl

_N = 8192
_D = 32
_R = 64
_EPS = 1e-5
_CLAMP = 64e-6


def _band(rlo, rhi, clo, chi, val=1.0):
    row = jax.lax.broadcasted_iota(jnp.int32, (128, 128), 0)
    col = jax.lax.broadcasted_iota(jnp.int32, (128, 128), 1)
    m = (row >= rlo) & (row < rhi) & (col >= clo) & (col < chi)
    return jnp.where(m, jnp.float32(val), jnp.float32(0.0))


def _padrow(v2, width=128):
    return jnp.pad(v2, ((0, 0), (0, width - v2.shape[1])))


def _body(z_ref, wq, bq, wk, bk, wv, bv, wo, bo, w1, b1, w2, b2,
          g1, be1, g2, be2, om, out_ref):
    f32 = jnp.float32
    dot = lambda a, b: jax.lax.dot_general(
        a, b, (((1,), (0,)), ((), ())), preferred_element_type=f32)
    dot_t = lambda a, b: jax.lax.dot_general(
        a, b, (((1,), (1,)), ((), ())), preferred_element_type=f32)

    Z = z_ref[...]
    zero = jnp.zeros((_D, _D), f32)

    W4 = jnp.concatenate([wq[...], wk[...], wv[...], zero], axis=0)
    b4 = jnp.concatenate(
        [bq[...], bk[...], bv[...], jnp.zeros((1, _D), f32)], axis=1)
    omega = om[...]
    zo = jnp.zeros((_D, _R), f32)
    G = jnp.concatenate([
        jnp.concatenate([omega, zo], axis=1),
        jnp.concatenate([zo, omega], axis=1),
        jnp.zeros((_R, 128), f32)], axis=0)
    H = _band(0, _D, 0, _R) + _band(_D, 2 * _D, _R, 128)
    E = _band(0, _D, 0, _D) * (jax.lax.broadcasted_iota(jnp.int32, (128, 128), 0)
                               == jax.lax.broadcasted_iota(jnp.int32, (128, 128), 1))
    A1 = _band(0, _D, 0, _D, 1.0 / _D)
    B = _band(_D, _D + 1, 0, _D)
    mask = _band(0, 1, 0, _D)[0:1, :]

    QKVV = dot_t(Z, W4) + b4
    NRM = dot(QKVV * QKVV, H)
    P = dot(QKVV, G)
    T = jnp.minimum(jax.lax.rsqrt(NRM), 1e6)
    PHI = jnp.exp(P * T)

    S_full = jax.lax.dot_general(
        PHI, QKVV, (((0,), (0,)), ((), ())), preferred_element_type=f32)
    S = S_full[_R:128, 2 * _D:3 * _D]
    SM = dot_t(S, wo[...])
    Ssum = jnp.sum(S, axis=1, keepdims=True)
    M2 = jnp.concatenate([
        SM, Ssum, jnp.zeros((_R, 128 - _D - 1), f32)], axis=1)
    M2 = jnp.concatenate([M2, jnp.zeros((_R, 128), f32)], axis=0)

    U = dot(PHI, M2)
    Dn = dot(U, B)
    t = (1.0 / jnp.maximum(Dn, _CLAMP)) * mask
    X = dot(Z, E[0:_D, :]) + U * t + _padrow(bo[...])

    SX = dot(X, A1)
    SXX = dot(X * X, A1)
    XN = (X - SX) * jax.lax.rsqrt(SXX - SX * SX + _EPS)

    g1p = _padrow(g1[...])
    XNg = XN * g1p
    b1x = dot_t(be1[...], w1[...]) + b1[...]
    W1pad = jnp.concatenate([w1[...], jnp.zeros((128, 96), f32)], axis=1)
    hid = jnp.maximum(dot_t(XNg, W1pad) + b1x, 0.0)
    Z1 = XNg + _padrow(be1[...])
    W2pad = jnp.concatenate([w2[...], jnp.zeros((96, 128), f32)], axis=0)
    Y = Z1 + dot_t(hid, W2pad) + _padrow(b2[...])

    SY = dot(Y, A1)
    SYY = dot(Y * Y, A1)
    YN = (Y - SY) * jax.lax.rsqrt(SYY - SY * SY + _EPS)

    G2 = E[:, 0:_D] * g2[...]
    out_ref[...] = dot(YN, G2) + be2[...]


@jax.jit
def kernel(Z, Wq, bq, Wk, bk, Wv, bv, Wo, bo, W1, b1, W2, b2,
           g1, beta1, g2, beta2, omega):
    row = lambda v: v.reshape(1, -1)
    args = (Z, Wq, row(bq), Wk, row(bk), Wv, row(bv), Wo, row(bo),
            W1, row(b1), W2, row(b2), row(g1), row(beta1), row(g2),
            row(beta2), omega)
    return pl.pallas_call(
        _body,
        out_shape=jax.ShapeDtypeStruct((_N, _D), jnp.float32),
    )(*args)


# speedup vs baseline: 3.1730x; 1.4685x over previous
import jax
import jax.numpy as jnp
from jax.experimental import pallas as pl

_N = 8192
_D = 32
_R = 64
_EPS = 1e-5
_CLAMP = 64e-6


def _body(zt_ref, wq, wk, wv, wo, w1t, w2, om, bq, bk, bv, bo, b1, b2,
          g1, be1, g2, be2, out_ref):
    f32 = jnp.float32
    dg = lambda a, b, dims: jax.lax.dot_general(
        a, b, (dims, ((), ())), preferred_element_type=f32)

    ZT = zt_ref[...]

    W4 = jnp.concatenate([wq[...], wk[...], wv[...]], axis=0)
    b4 = jnp.concatenate([bq[...], bk[...], bv[...]], axis=0)
    QKVV = dg(W4, ZT, ((1,), (0,))) + b4

    col = jax.lax.broadcasted_iota(jnp.int32, (2, 3 * _D), 1)
    rowi = jax.lax.broadcasted_iota(jnp.int32, (2, 3 * _D), 0)
    O2 = jnp.where((col >= rowi * _D) & (col < (rowi + 1) * _D),
                   jnp.float32(1.0), jnp.float32(0.0))
    N2 = dg(O2, QKVV * QKVV, ((1,), (0,)))
    T2 = jnp.minimum(jax.lax.rsqrt(N2), 1e6)
    Tb = jnp.concatenate([
        jnp.broadcast_to(T2[0:1, :], (_R, _N)),
        jnp.broadcast_to(T2[1:2, :], (_R, _N))], axis=0)

    OMT = om[...].T
    zo = jnp.zeros((_R, _D), f32)
    GT = jnp.concatenate([
        jnp.concatenate([OMT, zo, zo], axis=1),
        jnp.concatenate([zo, OMT, zo], axis=1)], axis=0)
    P = dg(GT, QKVV, ((1,), (0,)))
    PHI = jnp.exp(P * Tb)

    Sf = dg(PHI, QKVV, ((1,), (1,)))
    S = Sf[_R:128, 2 * _D:3 * _D]
    SWoT = dg(wo[...], S, ((1,), (1,)))
    ones32 = jnp.ones((1, _D), f32)
    SsT = dg(ones32, S, ((1,), (1,)))
    M2T = jnp.concatenate([
        jnp.concatenate([SWoT, SsT], axis=0),
        jnp.zeros((_D + 1, _R), f32)], axis=1)

    U = dg(M2T, PHI, ((1,), (0,)))
    t = 1.0 / jnp.maximum(U[_D:_D + 1, :], _CLAMP)
    X = ZT + U[0:_D, :] * t + bo[...]

    O32 = jnp.full((1, _D), 1.0 / _D, f32)
    mu = dg(O32, X, ((1,), (0,)))
    m2 = dg(O32, X * X, ((1,), (0,)))
    XN = (X - mu) * jax.lax.rsqrt(m2 - mu * mu + _EPS)
    Z1 = XN * g1[...] + be1[...]

    hid = jnp.maximum(dg(w1t[...], Z1, ((0,), (0,))) + b1[...], 0.0)
    Y = Z1 + dg(w2[...], hid, ((1,), (0,))) + b2[...]

    mu2 = dg(O32, Y, ((1,), (0,)))
    m22 = dg(O32, Y * Y, ((1,), (0,)))
    YN = (Y - mu2) * jax.lax.rsqrt(m22 - mu2 * mu2 + _EPS)
    out_ref[...] = YN * g2[...] + be2[...]


@jax.jit
def kernel(Z, Wq, bq, Wk, bk, Wv, bv, Wo, bo, W1, b1, W2, b2,
           g1, beta1, g2, beta2, omega):
    colv = lambda v: v.reshape(-1, 1)
    args = (Z.T, Wq, Wk, Wv, Wo, W1.T, W2, omega,
            colv(bq), colv(bk), colv(bv), colv(bo), colv(b1), colv(b2),
            colv(g1), colv(beta1), colv(g2), colv(beta2))
    out_t = pl.pallas_call(
        _body,
        out_shape=jax.ShapeDtypeStruct((_D, _N), jnp.float32),
    )(*args)
    return out_t.T


# speedup vs baseline: 6.4136x; 2.0213x over previous
import jax
import jax.numpy as jnp
from jax.experimental import pallas as pl

_N = 8192
_D = 32
_R = 64
_EPS = 1e-5
_CLAMP = 64e-6


def _body(zt_ref, wq, wk, wv, wo, w1t, w2, om, bq, bk, bv, bo, b1, b2,
          g1, be1, g2, be2, out_ref):
    f32 = jnp.float32
    dg = lambda a, b, dims: jax.lax.dot_general(
        a, b, (dims, ((), ())), preferred_element_type=f32)

    ZT = zt_ref[...]

    W4 = jnp.concatenate([wq[...], wk[...], wv[...]], axis=0)
    b4 = jnp.concatenate([bq[...], bk[...], bv[...]], axis=1).T
    QKVV = dg(W4, ZT, ((1,), (0,))) + b4

    col = jax.lax.broadcasted_iota(jnp.int32, (2, 3 * _D), 1)
    rowi = jax.lax.broadcasted_iota(jnp.int32, (2, 3 * _D), 0)
    O2 = jnp.where((col >= rowi * _D) & (col < (rowi + 1) * _D),
                   jnp.float32(1.0), jnp.float32(0.0))
    N2 = dg(O2, QKVV * QKVV, ((1,), (0,)))
    T2 = jnp.minimum(jax.lax.rsqrt(N2), 1e6)
    Tb = jnp.concatenate([
        jnp.broadcast_to(T2[0:1, :], (_R, _N)),
        jnp.broadcast_to(T2[1:2, :], (_R, _N))], axis=0)

    OMT = om[...].T
    zo = jnp.zeros((_R, _D), f32)
    GT = jnp.concatenate([
        jnp.concatenate([OMT, zo, zo], axis=1),
        jnp.concatenate([zo, OMT, zo], axis=1)], axis=0)
    P = dg(GT, QKVV, ((1,), (0,)))
    PHI = jnp.exp(P * Tb)

    Sf = dg(PHI, QKVV, ((1,), (1,)))
    S = Sf[_R:128, 2 * _D:3 * _D]
    SWoT = dg(wo[...], S, ((1,), (1,)))
    ones32 = jnp.ones((1, _D), f32)
    SsT = dg(ones32, S, ((1,), (1,)))
    M2T = jnp.concatenate([
        jnp.concatenate([SWoT, SsT], axis=0),
        jnp.zeros((_D + 1, _R), f32)], axis=1)

    U = dg(M2T, PHI, ((1,), (0,)))
    t = 1.0 / jnp.maximum(U[_D:_D + 1, :], _CLAMP)
    X = ZT + U[0:_D, :] * t + bo[...].T

    O32 = jnp.full((1, _D), 1.0 / _D, f32)
    mu = dg(O32, X, ((1,), (0,)))
    m2 = dg(O32, X * X, ((1,), (0,)))
    XN = (X - mu) * jax.lax.rsqrt(m2 - mu * mu + _EPS)
    Z1 = XN * g1[...].T + be1[...].T

    hid = jnp.maximum(dg(w1t[...], Z1, ((0,), (0,))) + b1[...].T, 0.0)
    Y = Z1 + dg(w2[...], hid, ((1,), (0,))) + b2[...].T

    mu2 = dg(O32, Y, ((1,), (0,)))
    m22 = dg(O32, Y * Y, ((1,), (0,)))
    YN = (Y - mu2) * jax.lax.rsqrt(m22 - mu2 * mu2 + _EPS)
    out_ref[...] = YN * g2[...].T + be2[...].T


@jax.jit
def kernel(Z, Wq, bq, Wk, bk, Wv, bv, Wo, bo, W1, b1, W2, b2,
           g1, beta1, g2, beta2, omega):
    rowv = lambda v: v.reshape(1, -1)
    args = (Z.T, Wq, Wk, Wv, Wo, W1.T, W2, omega,
            rowv(bq), rowv(bk), rowv(bv), rowv(bo), rowv(b1), rowv(b2),
            rowv(g1), rowv(beta1), rowv(g2), rowv(beta2))
    out_t = pl.pallas_call(
        _body,
        out_shape=jax.ShapeDtypeStruct((_D, _N), jnp.float32),
    )(*args)
    return out_t.T


# speedup vs baseline: 6.4231x; 1.0015x over previous
import jax
import jax.numpy as jnp
from jax.experimental import pallas as pl

_N = 8192
_D = 32
_R = 64
_EPS = 1e-5
_CLAMP = 64e-6


def _body(zt_ref, wq, wk, wv, wo, w1t, w2, om, bq, bk, bv, bo, b1, b2,
          g1, be1, g2, be2, out_ref):
    f32 = jnp.float32
    dg = lambda a, b, dims: jax.lax.dot_general(
        a, b, (dims, ((), ())), preferred_element_type=f32)

    ZT = zt_ref[...]

    W4 = jnp.concatenate([wq[...], wk[...], wv[...]], axis=0)
    b4 = jnp.concatenate([bq[...], bk[...], bv[...]], axis=1).T
    QKVV = dg(W4, ZT, ((1,), (0,))) + b4

    col = jax.lax.broadcasted_iota(jnp.int32, (2, 3 * _D), 1)
    rowi = jax.lax.broadcasted_iota(jnp.int32, (2, 3 * _D), 0)
    O2 = jnp.where((col >= rowi * _D) & (col < (rowi + 1) * _D),
                   jnp.float32(1.0), jnp.float32(0.0))
    N2 = dg(O2, QKVV * QKVV, ((1,), (0,)))
    T2 = jnp.minimum(jax.lax.rsqrt(N2), 1e6) * 1.4426950408889634
    T2b = jnp.concatenate([
        jnp.broadcast_to(T2[0:1, :], (_D, _N)),
        jnp.broadcast_to(T2[1:2, :], (_D, _N))], axis=0)
    QKn = QKVV[0:2 * _D, :] * T2b

    OMT = om[...].T
    zo = jnp.zeros((_R, _D), f32)
    GT = jnp.concatenate([
        jnp.concatenate([OMT, zo], axis=1),
        jnp.concatenate([zo, OMT], axis=1)], axis=0)
    P = dg(GT, QKn, ((1,), (0,)))
    PHI = jnp.exp2(P)

    Sf = dg(PHI, QKVV, ((1,), (1,)))
    S = Sf[_R:128, 2 * _D:3 * _D]
    SWoT = dg(wo[...], S, ((1,), (1,)))
    ones32 = jnp.ones((1, _D), f32)
    SsT = dg(ones32, S, ((1,), (1,)))
    M2T = jnp.concatenate([
        jnp.concatenate([SWoT, SsT], axis=0),
        jnp.zeros((_D + 1, _R), f32)], axis=1)

    U = dg(M2T, PHI, ((1,), (0,)))
    t = 1.0 / jnp.maximum(U[_D:_D + 1, :], _CLAMP)
    X = ZT + U[0:_D, :] * t + bo[...].T

    O32 = jnp.full((1, _D), 1.0 / _D, f32)
    mu = dg(O32, X, ((1,), (0,)))
    m2 = dg(O32, X * X, ((1,), (0,)))
    a1 = jax.lax.rsqrt(m2 - mu * mu + _EPS)
    XN = X * a1 - mu * a1
    Z1 = XN * g1[...].T + be1[...].T

    hid = jnp.maximum(dg(w1t[...], Z1, ((0,), (0,))) + b1[...].T, 0.0)
    Y = Z1 + dg(w2[...], hid, ((1,), (0,))) + b2[...].T

    mu2 = dg(O32, Y, ((1,), (0,)))
    m22 = dg(O32, Y * Y, ((1,), (0,)))
    a2 = jax.lax.rsqrt(m22 - mu2 * mu2 + _EPS)
    out_ref[...] = (Y * a2 - mu2 * a2) * g2[...].T + be2[...].T


@jax.jit
def kernel(Z, Wq, bq, Wk, bk, Wv, bv, Wo, bo, W1, b1, W2, b2,
           g1, beta1, g2, beta2, omega):
    rowv = lambda v: v.reshape(1, -1)
    args = (Z.T, Wq, Wk, Wv, Wo, W1.T, W2, omega,
            rowv(bq), rowv(bk), rowv(bv), rowv(bo), rowv(b1), rowv(b2),
            rowv(g1), rowv(beta1), rowv(g2), rowv(beta2))
    out_t = pl.pallas_call(
        _body,
        out_shape=jax.ShapeDtypeStruct((_D, _N), jnp.float32),
    )(*args)
    return out_t.T
